# Initial kernel scaffold; baseline (speedup 1.0000x reference)
import functools

import jax
import jax.numpy as jnp
from jax import lax
from jax.experimental import pallas as pl
from jax.experimental.pallas import tpu as pltpu
from jax.experimental.pallas import tpu_sc as plsc

_N_NODES = 10000
_N_EDGES = 320000
_D = 128
_N_GRAPHS = 512
_D_OUT = 64

_NC = 2
_NS = 16
_NW = _NC * _NS
_EPT = _N_EDGES // _NW
_CHUNK = 80
_NCH = _EPT // _CHUNK
_RPT = _N_NODES // _NS


def _spmm_body(table, srcr, dstr, zeros, out, src_v, dst_v, rows_v, agg_sh, sem):
    c = lax.axis_index("c")
    s = lax.axis_index("s")
    wid = s * _NC + c

    pltpu.sync_copy(srcr.at[wid], src_v)
    pltpu.sync_copy(dstr.at[wid], dst_v)

    rbase = s * _RPT
    pltpu.sync_copy(zeros.at[pl.ds(rbase, _RPT)], agg_sh.at[pl.ds(rbase, _RPT)])
    plsc.subcore_barrier()

    def body(j, carry):
        pltpu.async_copy(table.at[src_v.at[j]], rows_v, sem).wait()
        pltpu.sync_copy(rows_v, agg_sh.at[dst_v.at[j]], add=True)
        return carry

    lax.fori_loop(0, _NCH, body, 0)
    plsc.subcore_barrier()

    pltpu.sync_copy(agg_sh.at[pl.ds(rbase, _RPT)], out.at[c, pl.ds(rbase, _RPT)])


_spmm = functools.partial(
    pl.kernel,
    out_type=jax.ShapeDtypeStruct((_NC, _N_NODES, _D), jnp.float32),
    mesh=plsc.VectorSubcoreMesh(core_axis_name="c", subcore_axis_name="s"),
    scratch_types=[
        pltpu.VMEM((_NCH, _CHUNK), jnp.int32),
        pltpu.VMEM((_NCH, _CHUNK), jnp.int32),
        pltpu.VMEM((_CHUNK, _D), jnp.float32),
        pltpu.VMEM_SHARED((_N_NODES, _D), jnp.float32),
        pltpu.SemaphoreType.DMA,
    ],
)(_spmm_body)


_BLK = 1000
_NBLK = _N_NODES // _BLK


def _relu_kernel(x_ref, o_ref):
    o_ref[...] = jnp.maximum(x_ref[...], 0.0)


def _dense_kernel(x_ref, a0_ref, a1_ref, w_ref, b_ref, o_ref):
    h = x_ref[...] + a0_ref[...] + a1_ref[...]
    acc = jnp.dot(h, w_ref[...], preferred_element_type=jnp.float32)
    o_ref[...] = jnp.maximum(acc + b_ref[...], 0.0)


def _final_kernel(h_ref, a0_ref, a1_ref, w_ref, b_ref, bt_ref, wc_ref, bc_ref,
                  o_ref, acc_ref):
    i = pl.program_id(0)
    h = h_ref[...] + a0_ref[...] + a1_ref[...]
    h2 = jnp.maximum(
        jnp.dot(h, w_ref[...], preferred_element_type=jnp.float32) + b_ref[...], 0.0)
    onehot = (jnp.broadcast_to(bt_ref[...], (_BLK, _N_GRAPHS))
              == lax.broadcasted_iota(jnp.int32, (_BLK, _N_GRAPHS), 1)
              ).astype(jnp.float32)
    contrib = lax.dot_general(onehot, h2, (((0,), (0,)), ((), ())),
                              preferred_element_type=jnp.float32)

    @pl.when(i == 0)
    def _():
        acc_ref[...] = contrib

    @pl.when(i > 0)
    def _():
        acc_ref[...] = acc_ref[...] + contrib

    @pl.when(i == _NBLK - 1)
    def _():
        logits = jnp.dot(acc_ref[...], wc_ref[...],
                         preferred_element_type=jnp.float32) + bc_ref[...]
        o_ref[...] = jax.nn.sigmoid(logits)


def _relu(x):
    return pl.pallas_call(
        _relu_kernel,
        grid=(_NBLK,),
        in_specs=[pl.BlockSpec((_BLK, _D), lambda i: (i, 0))],
        out_specs=pl.BlockSpec((_BLK, _D), lambda i: (i, 0)),
        out_shape=jax.ShapeDtypeStruct((_N_NODES, _D), jnp.float32),
    )(x)


def _dense(x, a0, a1, W, b):
    return pl.pallas_call(
        _dense_kernel,
        grid=(_NBLK,),
        in_specs=[
            pl.BlockSpec((_BLK, _D), lambda i: (i, 0)),
            pl.BlockSpec((_BLK, _D), lambda i: (i, 0)),
            pl.BlockSpec((_BLK, _D), lambda i: (i, 0)),
            pl.BlockSpec((_D, _D), lambda i: (0, 0)),
            pl.BlockSpec((1, _D), lambda i: (0, 0)),
        ],
        out_specs=pl.BlockSpec((_BLK, _D), lambda i: (i, 0)),
        out_shape=jax.ShapeDtypeStruct((_N_NODES, _D), jnp.float32),
    )(x, a0, a1, W, b.reshape(1, _D))


def _final(h1, a0, a1, W2, b2, batch, Wc, bc):
    return pl.pallas_call(
        _final_kernel,
        grid=(_NBLK,),
        in_specs=[
            pl.BlockSpec((_BLK, _D), lambda i: (i, 0)),
            pl.BlockSpec((_BLK, _D), lambda i: (i, 0)),
            pl.BlockSpec((_BLK, _D), lambda i: (i, 0)),
            pl.BlockSpec((_D, _D), lambda i: (0, 0)),
            pl.BlockSpec((1, _D), lambda i: (0, 0)),
            pl.BlockSpec((_BLK, 1), lambda i: (i, 0)),
            pl.BlockSpec((_D, _D_OUT), lambda i: (0, 0)),
            pl.BlockSpec((1, _D_OUT), lambda i: (0, 0)),
        ],
        out_specs=pl.BlockSpec((_N_GRAPHS, _D_OUT), lambda i: (0, 0)),
        out_shape=jax.ShapeDtypeStruct((_N_GRAPHS, _D_OUT), jnp.float32),
        scratch_shapes=[pltpu.VMEM((_N_GRAPHS, _D), jnp.float32)],
    )(h1, a0, a1, W2, b2.reshape(1, _D), batch.astype(jnp.int32).reshape(_N_NODES, 1),
      Wc, bc.reshape(1, _D_OUT))


def kernel(x, edge_index, batch, W1, b1, W2, b2, Wc, bc):
    srcr = edge_index[0].astype(jnp.int32).reshape(_NW, _NCH, _CHUNK)
    dstr = edge_index[1].astype(jnp.int32).reshape(_NW, _NCH, _CHUNK)
    zeros = jnp.zeros((_N_NODES, _D), jnp.float32)

    r0 = _relu(x)
    agg1 = _spmm(r0, srcr, dstr, zeros)
    h1 = _dense(x, agg1[0], agg1[1], W1, b1)
    agg2 = _spmm(h1, srcr, dstr, zeros)
    return _final(h1, agg2[0], agg2[1], W2, b2, batch, Wc, bc)


# speedup vs baseline: 6.8382x; 6.8382x over previous
import functools

import jax
import jax.numpy as jnp
from jax import lax
from jax.experimental import pallas as pl
from jax.experimental.pallas import tpu as pltpu
from jax.experimental.pallas import tpu_sc as plsc

_N_NODES = 10000
_N_EDGES = 320000
_D = 128
_N_GRAPHS = 512
_D_OUT = 64

_NC = 2
_NS = 16
_NW = _NC * _NS
_EPT = _N_EDGES // _NW
_CHUNK = 80
_NCH = _EPT // _CHUNK
_RPT = 624
_RTAIL = _N_NODES - _NS * _RPT


def _spmm_body(table, srcr, dstr, zeros, out, src_v, dst_v, rows_v, agg_sh, sem):
    c = lax.axis_index("c")
    s = lax.axis_index("s")
    wid = s * _NC + c

    pltpu.sync_copy(srcr.at[wid], src_v)
    pltpu.sync_copy(dstr.at[wid], dst_v)

    rbase = s * _RPT
    pltpu.sync_copy(zeros.at[pl.ds(rbase, _RPT)], agg_sh.at[pl.ds(rbase, _RPT)])

    @pl.when(s == _NS - 1)
    def _():
        pltpu.sync_copy(zeros.at[pl.ds(_NS * _RPT, _RTAIL)],
                        agg_sh.at[pl.ds(_NS * _RPT, _RTAIL)])

    plsc.subcore_barrier()

    def body(j, carry):
        pltpu.async_copy(table.at[src_v.at[j]], rows_v, sem).wait()
        pltpu.sync_copy(rows_v, agg_sh.at[dst_v.at[j]], add=True)
        return carry

    lax.fori_loop(0, _NCH, body, 0)
    plsc.subcore_barrier()

    pltpu.sync_copy(agg_sh.at[pl.ds(rbase, _RPT)], out.at[c, pl.ds(rbase, _RPT)])

    @pl.when(s == _NS - 1)
    def _():
        pltpu.sync_copy(agg_sh.at[pl.ds(_NS * _RPT, _RTAIL)],
                        out.at[c, pl.ds(_NS * _RPT, _RTAIL)])


_spmm = functools.partial(
    pl.kernel,
    out_type=jax.ShapeDtypeStruct((_NC, _N_NODES, _D), jnp.float32),
    mesh=plsc.VectorSubcoreMesh(core_axis_name="c", subcore_axis_name="s"),
    scratch_types=[
        pltpu.VMEM((_NCH, _CHUNK), jnp.int32),
        pltpu.VMEM((_NCH, _CHUNK), jnp.int32),
        pltpu.VMEM((_CHUNK, _D), jnp.float32),
        pltpu.VMEM_SHARED((_N_NODES, _D), jnp.float32),
        pltpu.SemaphoreType.DMA,
    ],
)(_spmm_body)


_BLK = 1000
_NBLK = _N_NODES // _BLK


def _relu_kernel(x_ref, o_ref):
    o_ref[...] = jnp.maximum(x_ref[...], 0.0)


def _dense_kernel(x_ref, a0_ref, a1_ref, w_ref, b_ref, o_ref):
    h = x_ref[...] + a0_ref[...] + a1_ref[...]
    acc = jnp.dot(h, w_ref[...], preferred_element_type=jnp.float32)
    o_ref[...] = jnp.maximum(acc + b_ref[...], 0.0)


def _final_kernel(h_ref, a0_ref, a1_ref, w_ref, b_ref, bt_ref, wc_ref, bc_ref,
                  o_ref, acc_ref):
    i = pl.program_id(0)
    h = h_ref[...] + a0_ref[...] + a1_ref[...]
    h2 = jnp.maximum(
        jnp.dot(h, w_ref[...], preferred_element_type=jnp.float32) + b_ref[...], 0.0)
    onehot = (jnp.broadcast_to(bt_ref[...], (_BLK, _N_GRAPHS))
              == lax.broadcasted_iota(jnp.int32, (_BLK, _N_GRAPHS), 1)
              ).astype(jnp.float32)
    contrib = lax.dot_general(onehot, h2, (((0,), (0,)), ((), ())),
                              preferred_element_type=jnp.float32)

    @pl.when(i == 0)
    def _():
        acc_ref[...] = contrib

    @pl.when(i > 0)
    def _():
        acc_ref[...] = acc_ref[...] + contrib

    @pl.when(i == _NBLK - 1)
    def _():
        logits = jnp.dot(acc_ref[...], wc_ref[...],
                         preferred_element_type=jnp.float32) + bc_ref[...]
        o_ref[...] = jax.nn.sigmoid(logits)


def _relu(x):
    return pl.pallas_call(
        _relu_kernel,
        grid=(_NBLK,),
        in_specs=[pl.BlockSpec((_BLK, _D), lambda i: (i, 0))],
        out_specs=pl.BlockSpec((_BLK, _D), lambda i: (i, 0)),
        out_shape=jax.ShapeDtypeStruct((_N_NODES, _D), jnp.float32),
    )(x)


def _dense(x, a0, a1, W, b):
    return pl.pallas_call(
        _dense_kernel,
        grid=(_NBLK,),
        in_specs=[
            pl.BlockSpec((_BLK, _D), lambda i: (i, 0)),
            pl.BlockSpec((_BLK, _D), lambda i: (i, 0)),
            pl.BlockSpec((_BLK, _D), lambda i: (i, 0)),
            pl.BlockSpec((_D, _D), lambda i: (0, 0)),
            pl.BlockSpec((1, _D), lambda i: (0, 0)),
        ],
        out_specs=pl.BlockSpec((_BLK, _D), lambda i: (i, 0)),
        out_shape=jax.ShapeDtypeStruct((_N_NODES, _D), jnp.float32),
    )(x, a0, a1, W, b.reshape(1, _D))


def _final(h1, a0, a1, W2, b2, batch, Wc, bc):
    return pl.pallas_call(
        _final_kernel,
        grid=(_NBLK,),
        in_specs=[
            pl.BlockSpec((_BLK, _D), lambda i: (i, 0)),
            pl.BlockSpec((_BLK, _D), lambda i: (i, 0)),
            pl.BlockSpec((_BLK, _D), lambda i: (i, 0)),
            pl.BlockSpec((_D, _D), lambda i: (0, 0)),
            pl.BlockSpec((1, _D), lambda i: (0, 0)),
            pl.BlockSpec((_BLK, 1), lambda i: (i, 0)),
            pl.BlockSpec((_D, _D_OUT), lambda i: (0, 0)),
            pl.BlockSpec((1, _D_OUT), lambda i: (0, 0)),
        ],
        out_specs=pl.BlockSpec((_N_GRAPHS, _D_OUT), lambda i: (0, 0)),
        out_shape=jax.ShapeDtypeStruct((_N_GRAPHS, _D_OUT), jnp.float32),
        scratch_shapes=[pltpu.VMEM((_N_GRAPHS, _D), jnp.float32)],
    )(h1, a0, a1, W2, b2.reshape(1, _D), batch.astype(jnp.int32).reshape(_N_NODES, 1),
      Wc, bc.reshape(1, _D_OUT))


def kernel(x, edge_index, batch, W1, b1, W2, b2, Wc, bc):
    srcr = edge_index[0].astype(jnp.int32).reshape(_NW, _NCH, _CHUNK)
    dstr = edge_index[1].astype(jnp.int32).reshape(_NW, _NCH, _CHUNK)
    zeros = jnp.zeros((_N_NODES, _D), jnp.float32)

    r0 = _relu(x)
    agg1 = _spmm(r0, srcr, dstr, zeros)
    h1 = _dense(x, agg1[0], agg1[1], W1, b1)
    agg2 = _spmm(h1, srcr, dstr, zeros)
    return _final(h1, agg2[0], agg2[1], W2, b2, batch, Wc, bc)


# speedup vs baseline: 9.7173x; 1.4210x over previous
import functools

import jax
import jax.numpy as jnp
from jax import lax
from jax.experimental import pallas as pl
from jax.experimental.pallas import tpu as pltpu
from jax.experimental.pallas import tpu_sc as plsc

_N_NODES = 10000
_N_EDGES = 320000
_D = 128
_N_GRAPHS = 512
_D_OUT = 64

_NC = 2
_NS = 16
_NW = _NC * _NS
_CHUNK = 125
_NCHT = _N_EDGES // _CHUNK
_PH = 40
_RPT = 624
_RTAIL = _N_NODES - _NS * _RPT


def _spmm_body(table, srcr, dstr, zeros, out, src_v, dst_v, rows0_v, rows1_v,
               agg_sh, sem0, sem1):
    c = lax.axis_index("c")
    s = lax.axis_index("s")
    wid = s * _NC + c

    rbase = s * _RPT
    pltpu.sync_copy(zeros.at[pl.ds(rbase, _RPT)], agg_sh.at[pl.ds(rbase, _RPT)])

    @pl.when(s == _NS - 1)
    def _():
        pltpu.sync_copy(zeros.at[pl.ds(_NS * _RPT, _RTAIL)],
                        agg_sh.at[pl.ds(_NS * _RPT, _RTAIL)])

    plsc.subcore_barrier()

    def _step(j, cur_v, csem, nxt_v, nsem):
        pltpu.make_async_copy(table.at[src_v.at[j]], cur_v, csem).wait()

        @pl.when(j + 1 < _PH)
        def _():
            pltpu.async_copy(table.at[src_v.at[j + 1]], nxt_v, nsem)

        pltpu.sync_copy(cur_v, agg_sh.at[dst_v.at[j]], add=True)

    def _body(j, carry):
        @pl.when(j % 2 == 0)
        def _():
            _step(j, rows0_v, sem0, rows1_v, sem1)

        @pl.when(j % 2 == 1)
        def _():
            _step(j, rows1_v, sem1, rows0_v, sem0)

        return carry

    for phase in range(2):
        pbase = wid * (2 * _PH) + phase * _PH
        pltpu.sync_copy(srcr.at[pl.ds(pbase, _PH)], src_v)
        pltpu.sync_copy(dstr.at[pl.ds(pbase, _PH)], dst_v)
        pltpu.async_copy(table.at[src_v.at[0]], rows0_v, sem0)
        lax.fori_loop(0, _PH, _body, 0)

    plsc.subcore_barrier()

    pltpu.sync_copy(agg_sh.at[pl.ds(rbase, _RPT)], out.at[c, pl.ds(rbase, _RPT)])

    @pl.when(s == _NS - 1)
    def _():
        pltpu.sync_copy(agg_sh.at[pl.ds(_NS * _RPT, _RTAIL)],
                        out.at[c, pl.ds(_NS * _RPT, _RTAIL)])


_spmm = functools.partial(
    pl.kernel,
    out_type=jax.ShapeDtypeStruct((_NC, _N_NODES, _D), jnp.float32),
    mesh=plsc.VectorSubcoreMesh(core_axis_name="c", subcore_axis_name="s"),
    scratch_types=[
        pltpu.VMEM((_PH, _CHUNK), jnp.int32),
        pltpu.VMEM((_PH, _CHUNK), jnp.int32),
        pltpu.VMEM((_CHUNK, _D), jnp.float32),
        pltpu.VMEM((_CHUNK, _D), jnp.float32),
        pltpu.VMEM_SHARED((_N_NODES, _D), jnp.float32),
        pltpu.SemaphoreType.DMA,
        pltpu.SemaphoreType.DMA,
    ],
)(_spmm_body)


_BLK = 1000
_NBLK = _N_NODES // _BLK


def _relu_kernel(x_ref, o_ref):
    o_ref[...] = jnp.maximum(x_ref[...], 0.0)


def _dense_kernel(x_ref, a0_ref, a1_ref, w_ref, b_ref, o_ref):
    h = x_ref[...] + a0_ref[...] + a1_ref[...]
    acc = jnp.dot(h, w_ref[...], preferred_element_type=jnp.float32)
    o_ref[...] = jnp.maximum(acc + b_ref[...], 0.0)


def _final_kernel(h_ref, a0_ref, a1_ref, w_ref, b_ref, bt_ref, wc_ref, bc_ref,
                  o_ref, acc_ref):
    i = pl.program_id(0)
    h = h_ref[...] + a0_ref[...] + a1_ref[...]
    h2 = jnp.maximum(
        jnp.dot(h, w_ref[...], preferred_element_type=jnp.float32) + b_ref[...], 0.0)
    onehot = (jnp.broadcast_to(bt_ref[...], (_BLK, _N_GRAPHS))
              == lax.broadcasted_iota(jnp.int32, (_BLK, _N_GRAPHS), 1)
              ).astype(jnp.float32)
    contrib = lax.dot_general(onehot, h2, (((0,), (0,)), ((), ())),
                              preferred_element_type=jnp.float32)

    @pl.when(i == 0)
    def _():
        acc_ref[...] = contrib

    @pl.when(i > 0)
    def _():
        acc_ref[...] = acc_ref[...] + contrib

    @pl.when(i == _NBLK - 1)
    def _():
        logits = jnp.dot(acc_ref[...], wc_ref[...],
                         preferred_element_type=jnp.float32) + bc_ref[...]
        o_ref[...] = jax.nn.sigmoid(logits)


def _relu(x):
    return pl.pallas_call(
        _relu_kernel,
        grid=(_NBLK,),
        in_specs=[pl.BlockSpec((_BLK, _D), lambda i: (i, 0))],
        out_specs=pl.BlockSpec((_BLK, _D), lambda i: (i, 0)),
        out_shape=jax.ShapeDtypeStruct((_N_NODES, _D), jnp.float32),
    )(x)


def _dense(x, a0, a1, W, b):
    return pl.pallas_call(
        _dense_kernel,
        grid=(_NBLK,),
        in_specs=[
            pl.BlockSpec((_BLK, _D), lambda i: (i, 0)),
            pl.BlockSpec((_BLK, _D), lambda i: (i, 0)),
            pl.BlockSpec((_BLK, _D), lambda i: (i, 0)),
            pl.BlockSpec((_D, _D), lambda i: (0, 0)),
            pl.BlockSpec((1, _D), lambda i: (0, 0)),
        ],
        out_specs=pl.BlockSpec((_BLK, _D), lambda i: (i, 0)),
        out_shape=jax.ShapeDtypeStruct((_N_NODES, _D), jnp.float32),
    )(x, a0, a1, W, b.reshape(1, _D))


def _final(h1, a0, a1, W2, b2, batch, Wc, bc):
    return pl.pallas_call(
        _final_kernel,
        grid=(_NBLK,),
        in_specs=[
            pl.BlockSpec((_BLK, _D), lambda i: (i, 0)),
            pl.BlockSpec((_BLK, _D), lambda i: (i, 0)),
            pl.BlockSpec((_BLK, _D), lambda i: (i, 0)),
            pl.BlockSpec((_D, _D), lambda i: (0, 0)),
            pl.BlockSpec((1, _D), lambda i: (0, 0)),
            pl.BlockSpec((_BLK, 1), lambda i: (i, 0)),
            pl.BlockSpec((_D, _D_OUT), lambda i: (0, 0)),
            pl.BlockSpec((1, _D_OUT), lambda i: (0, 0)),
        ],
        out_specs=pl.BlockSpec((_N_GRAPHS, _D_OUT), lambda i: (0, 0)),
        out_shape=jax.ShapeDtypeStruct((_N_GRAPHS, _D_OUT), jnp.float32),
        scratch_shapes=[pltpu.VMEM((_N_GRAPHS, _D), jnp.float32)],
    )(h1, a0, a1, W2, b2.reshape(1, _D), batch.astype(jnp.int32).reshape(_N_NODES, 1),
      Wc, bc.reshape(1, _D_OUT))


def kernel(x, edge_index, batch, W1, b1, W2, b2, Wc, bc):
    srcr = edge_index[0].astype(jnp.int32).reshape(_NCHT, _CHUNK)
    dstr = edge_index[1].astype(jnp.int32).reshape(_NCHT, _CHUNK)
    zeros = jnp.zeros((_N_NODES, _D), jnp.float32)

    r0 = _relu(x)
    agg1 = _spmm(r0, srcr, dstr, zeros)
    h1 = _dense(x, agg1[0], agg1[1], W1, b1)
    agg2 = _spmm(h1, srcr, dstr, zeros)
    return _final(h1, agg2[0], agg2[1], W2, b2, batch, Wc, bc)
